# Initial kernel scaffold; baseline (speedup 1.0000x reference)
import jax
import jax.numpy as jnp
from jax.experimental import pallas as pl


def kernel(x, edge_index, lin1_w, lin1_b, coeffs0, coeffs1, lin2_w, lin2_b):
    raise NotImplementedError("write your pallas kernel here")


# speedup vs baseline: 9.1419x; 9.1419x over previous
import functools

import jax
import jax.numpy as jnp
from jax import lax
from jax.experimental import pallas as pl
from jax.experimental.pallas import tpu as pltpu
from jax.experimental.pallas import tpu_sc as plsc

N_NODES = 10000
NPAD = 10240
N_EDGES = 320000
D_FEAT = 128
K_ORDER = 10
A_J = 0.5
B_J = 0.5

CHUNK = 128
NCH = N_EDGES // CHUNK
NCH_SC = NCH // 2
NTILES = 16
ROWS_PER_TILE = NPAD // NTILES

_mesh = plsc.VectorSubcoreMesh(core_axis_name="c", subcore_axis_name="s")


@functools.partial(
    pl.kernel, mesh=_mesh,
    out_type=jax.ShapeDtypeStruct((2 * NPAD, D_FEAT), jnp.float32),
    scratch_types=[
        pltpu.VMEM_SHARED((NPAD, D_FEAT), jnp.float32),
        pltpu.VMEM((CHUNK,), jnp.int32),
        pltpu.VMEM((CHUNK,), jnp.int32),
        pltpu.VMEM((CHUNK, D_FEAT), jnp.float32),
        pltpu.VMEM((CHUNK, D_FEAT), jnp.float32),
        pltpu.SemaphoreType.DMA,
    ],
)
def _prop(z_hbm, row_hbm, col_hbm, part_hbm, acc_sh, idxg, idxs, gbuf, zbuf, sem):
    c = lax.axis_index("c")
    t = lax.axis_index("s")
    base_rows = t * ROWS_PER_TILE

    z16 = jnp.zeros((16,), jnp.float32)

    def zb(i, _):
        for j in range(D_FEAT // 16):
            zbuf[i, pl.ds(j * 16, 16)] = z16
        return 0

    lax.fori_loop(0, CHUNK, zb, 0)
    for j in range(ROWS_PER_TILE // CHUNK):
        pltpu.sync_copy(zbuf, acc_sh.at[pl.ds(base_rows + j * CHUNK, CHUNK)])
    plsc.subcore_barrier()

    nt = (NCH_SC - t + NTILES - 1) // NTILES

    def chunk_body(jj, _):
        base = (c * NCH_SC + jj * NTILES + t) * CHUNK
        pltpu.sync_copy(row_hbm.at[pl.ds(base, CHUNK)], idxg)
        pltpu.sync_copy(col_hbm.at[pl.ds(base, CHUNK)], idxs)
        pltpu.async_copy(z_hbm.at[idxg], gbuf, sem).wait()
        pltpu.sync_copy(gbuf, acc_sh.at[idxs], add=True)
        return 0

    lax.fori_loop(0, nt, chunk_body, 0)
    plsc.subcore_barrier()
    pltpu.sync_copy(acc_sh.at[pl.ds(base_rows, ROWS_PER_TILE)],
                    part_hbm.at[pl.ds(c * NPAD + base_rows, ROWS_PER_TILE)])


def _linear(x, w, b, relu):
    m, kdim = x.shape
    dout = w.shape[0]
    bm = 1000

    def body(x_ref, w_ref, b_ref, o_ref):
        acc = lax.dot_general(x_ref[...], w_ref[...], (((1,), (1,)), ((), ())),
                              preferred_element_type=jnp.float32)
        acc = acc + b_ref[...][None, :]
        if relu:
            acc = jnp.maximum(acc, 0.0)
        o_ref[...] = acc

    return pl.pallas_call(
        body,
        grid=(m // bm,),
        in_specs=[
            pl.BlockSpec((bm, kdim), lambda i: (i, 0)),
            pl.BlockSpec((dout, kdim), lambda i: (0, 0)),
            pl.BlockSpec((dout,), lambda i: (0,)),
        ],
        out_specs=pl.BlockSpec((bm, dout), lambda i: (i, 0)),
        out_shape=jax.ShapeDtypeStruct((m, dout), jnp.float32),
    )(x, w, b)


_NB = NPAD // 1024


def _step(part, zc, zp, acc, dinv2e, ck, a_n, c_n):
    bm = 1024

    def body(ck_ref, p0_ref, p1_ref, zc_ref, zp_ref, acc_ref, d_ref,
             z2_ref, accn_ref):
        tt = (p0_ref[...] + p1_ref[...] + zc_ref[...]) * d_ref[...]
        z2 = a_n * tt - c_n * zp_ref[...]
        z2_ref[...] = z2
        accn_ref[...] = acc_ref[...] + ck_ref[0, 0] * z2

    return pl.pallas_call(
        body,
        grid=(_NB,),
        in_specs=[
            pl.BlockSpec(memory_space=pltpu.SMEM),
            pl.BlockSpec((bm, D_FEAT), lambda i: (i, 0)),
            pl.BlockSpec((bm, D_FEAT), lambda i: (i + _NB, 0)),
            pl.BlockSpec((bm, D_FEAT), lambda i: (i, 0)),
            pl.BlockSpec((bm, D_FEAT), lambda i: (i, 0)),
            pl.BlockSpec((bm, D_FEAT), lambda i: (i, 0)),
            pl.BlockSpec((bm, 1), lambda i: (i, 0)),
        ],
        out_specs=[
            pl.BlockSpec((bm, D_FEAT), lambda i: (i, 0)),
            pl.BlockSpec((bm, D_FEAT), lambda i: (i, 0)),
        ],
        out_shape=[
            jax.ShapeDtypeStruct((NPAD, D_FEAT), jnp.float32),
            jax.ShapeDtypeStruct((NPAD, D_FEAT), jnp.float32),
        ],
        input_output_aliases={5: 1},
    )(ck, part, part, zc, zp, acc, dinv2e)


def kernel(x, edge_index, lin1_w, lin1_b, coeffs0, coeffs1, lin2_w, lin2_b):
    row = edge_index[0].astype(jnp.int32)
    col = edge_index[1].astype(jnp.int32)

    ones_mat = jnp.ones((NPAD, D_FEAT), jnp.float32)
    dp = _prop(ones_mat, row, col)
    deg = dp[:NPAD, 0] + dp[NPAD:, 0] + 1.0
    deg = jnp.where(jnp.arange(NPAD) < N_NODES, deg, 1.0)
    dinve = lax.rsqrt(deg)[:, None]
    dsqrte = jnp.sqrt(deg)[:, None]
    dinv2e = (1.0 / deg)[:, None]

    h = _linear(x, lin1_w, lin1_b, relu=True)
    hs = jnp.pad(h, ((0, NPAD - N_NODES), (0, 0)))

    for coeffs in (coeffs0, coeffs1):
        z0 = dinve * hs
        acc = coeffs[0] * z0
        zc, zp = z0, z0
        for k in range(1, K_ORDER + 1):
            part = _prop(zc, row, col)
            if k == 1:
                a_n, c_n = 1.0, 0.0
            else:
                n = k - 1
                a_n = (2 * n + A_J + B_J + 1) * (2 * n + A_J + B_J + 2) / (
                    2 * (n + 1) * (n + A_J + B_J + 1))
                c_n = (n + A_J) * (n + B_J) * (2 * n + A_J + B_J + 2) / (
                    (n + 1) * (n + A_J + B_J + 1) * (2 * n + A_J + B_J))
            znew, acc = _step(part, zc, zp, acc, dinv2e,
                              coeffs[k].reshape(1, 1), a_n, c_n)
            zp, zc = zc, znew
        hs = jnp.maximum(dsqrte * acc, 0.0)

    out = _linear(hs[:N_NODES], lin2_w, lin2_b, relu=False)
    return out


# speedup vs baseline: 11.2069x; 1.2259x over previous
import functools

import jax
import jax.numpy as jnp
from jax import lax
from jax.experimental import pallas as pl
from jax.experimental.pallas import tpu as pltpu
from jax.experimental.pallas import tpu_sc as plsc

N_NODES = 10000
NPAD = 10240
N_EDGES = 320000
D_FEAT = 128
K_ORDER = 10
A_J = 0.5
B_J = 0.5

CHUNK = 64
NGROUP = 4
E_PAD = 327680
NCH_SC = E_PAD // CHUNK // 2
NTILES = 16
CH_PER_TILE = NCH_SC // NTILES
NGROUPS = CH_PER_TILE // NGROUP
ROWS_PER_TILE = NPAD // NTILES

_mesh = plsc.VectorSubcoreMesh(core_axis_name="c", subcore_axis_name="s")


@functools.partial(
    pl.kernel, mesh=_mesh,
    out_type=jax.ShapeDtypeStruct((2 * NPAD, D_FEAT), jnp.float32),
    scratch_types=[
        pltpu.VMEM_SHARED((NPAD, D_FEAT), jnp.float32),
        *([pltpu.VMEM((CHUNK,), jnp.int32)] * NGROUP),
        *([pltpu.VMEM((CHUNK,), jnp.int32)] * NGROUP),
        *([pltpu.VMEM((CHUNK, D_FEAT), jnp.float32)] * NGROUP),
        pltpu.SemaphoreType.DMA,
        pltpu.SemaphoreType.DMA,
    ],
)
def _prop(z_hbm, row_hbm, col_hbm, part_hbm, acc_sh,
          ig0, ig1, ig2, ig3, is0, is1, is2, is3, gb0, gb1, gb2, gb3,
          sem_g, sem_s):
    idxg = (ig0, ig1, ig2, ig3)
    idxs = (is0, is1, is2, is3)
    gbuf = (gb0, gb1, gb2, gb3)
    c = lax.axis_index("c")
    t = lax.axis_index("s")
    base_rows = t * ROWS_PER_TILE

    z16 = jnp.zeros((16,), jnp.float32)

    def zb(i, _):
        for j in range(D_FEAT // 16):
            gb0[i, pl.ds(j * 16, 16)] = z16
        return 0

    lax.fori_loop(0, CHUNK, zb, 0)
    for j in range(ROWS_PER_TILE // CHUNK):
        pltpu.sync_copy(gb0, acc_sh.at[pl.ds(base_rows + j * CHUNK, CHUNK)])
    plsc.subcore_barrier()

    def chunk_base(jj):
        return (c * NCH_SC + jj * NTILES + t) * CHUNK

    def load_idxg(g):
        for b in range(NGROUP):
            pltpu.sync_copy(row_hbm.at[pl.ds(chunk_base(g * NGROUP + b), CHUNK)],
                            idxg[b])

    def load_idxs(g, b):
        pltpu.sync_copy(col_hbm.at[pl.ds(chunk_base(g * NGROUP + b), CHUNK)],
                        idxs[b])

    load_idxg(0)
    for b in range(NGROUP):
        load_idxs(0, b)
        pltpu.make_async_copy(z_hbm.at[idxg[b]], gbuf[b], sem_g).start()

    def body(g, _):
        for b in range(NGROUP):
            pltpu.make_async_copy(z_hbm.at[idxg[b]], gbuf[b], sem_g).wait()
        for b in range(NGROUP):
            pltpu.make_async_copy(gbuf[b], acc_sh.at[idxs[b]],
                                  sem_s).start(add=True)

        @pl.when(g + 1 < NGROUPS)
        def _prefetch():
            load_idxg(g + 1)
            for b in range(NGROUP):
                pltpu.make_async_copy(gbuf[b], acc_sh.at[idxs[b]],
                                      sem_s).wait()
                load_idxs(g + 1, b)
                pltpu.make_async_copy(z_hbm.at[idxg[b]], gbuf[b],
                                      sem_g).start()

        @pl.when(g + 1 >= NGROUPS)
        def _drain():
            for b in range(NGROUP):
                pltpu.make_async_copy(gbuf[b], acc_sh.at[idxs[b]],
                                      sem_s).wait()

        return 0

    lax.fori_loop(0, NGROUPS, body, 0)
    plsc.subcore_barrier()
    pltpu.sync_copy(acc_sh.at[pl.ds(base_rows, ROWS_PER_TILE)],
                    part_hbm.at[pl.ds(c * NPAD + base_rows, ROWS_PER_TILE)])


def _linear(x, w, b, relu):
    m, kdim = x.shape
    dout = w.shape[0]
    bm = 1000

    def body(x_ref, w_ref, b_ref, o_ref):
        acc = lax.dot_general(x_ref[...], w_ref[...], (((1,), (1,)), ((), ())),
                              preferred_element_type=jnp.float32)
        acc = acc + b_ref[...][None, :]
        if relu:
            acc = jnp.maximum(acc, 0.0)
        o_ref[...] = acc

    return pl.pallas_call(
        body,
        grid=(m // bm,),
        in_specs=[
            pl.BlockSpec((bm, kdim), lambda i: (i, 0)),
            pl.BlockSpec((dout, kdim), lambda i: (0, 0)),
            pl.BlockSpec((dout,), lambda i: (0,)),
        ],
        out_specs=pl.BlockSpec((bm, dout), lambda i: (i, 0)),
        out_shape=jax.ShapeDtypeStruct((m, dout), jnp.float32),
    )(x, w, b)


_NB = NPAD // 1024


def _step(part, zc, zp, acc, dinv2e, ck, a_n, c_n):
    bm = 1024

    def body(ck_ref, p0_ref, p1_ref, zc_ref, zp_ref, acc_ref, d_ref,
             z2_ref, accn_ref):
        tt = (p0_ref[...] + p1_ref[...] + zc_ref[...]) * d_ref[...]
        z2 = a_n * tt - c_n * zp_ref[...]
        z2_ref[...] = z2
        accn_ref[...] = acc_ref[...] + ck_ref[0, 0] * z2

    return pl.pallas_call(
        body,
        grid=(_NB,),
        in_specs=[
            pl.BlockSpec(memory_space=pltpu.SMEM),
            pl.BlockSpec((bm, D_FEAT), lambda i: (i, 0)),
            pl.BlockSpec((bm, D_FEAT), lambda i: (i + _NB, 0)),
            pl.BlockSpec((bm, D_FEAT), lambda i: (i, 0)),
            pl.BlockSpec((bm, D_FEAT), lambda i: (i, 0)),
            pl.BlockSpec((bm, D_FEAT), lambda i: (i, 0)),
            pl.BlockSpec((bm, 1), lambda i: (i, 0)),
        ],
        out_specs=[
            pl.BlockSpec((bm, D_FEAT), lambda i: (i, 0)),
            pl.BlockSpec((bm, D_FEAT), lambda i: (i, 0)),
        ],
        out_shape=[
            jax.ShapeDtypeStruct((NPAD, D_FEAT), jnp.float32),
            jax.ShapeDtypeStruct((NPAD, D_FEAT), jnp.float32),
        ],
        input_output_aliases={5: 1},
    )(ck, part, part, zc, zp, acc, dinv2e)


def kernel(x, edge_index, lin1_w, lin1_b, coeffs0, coeffs1, lin2_w, lin2_b):
    row = edge_index[0].astype(jnp.int32)
    col = edge_index[1].astype(jnp.int32)
    padv = N_NODES + (jnp.arange(E_PAD - N_EDGES, dtype=jnp.int32)
                      % (NPAD - N_NODES))
    row = jnp.concatenate([row, padv])
    col = jnp.concatenate([col, padv])

    ones_mat = jnp.ones((NPAD, D_FEAT), jnp.float32)
    dp = _prop(ones_mat, row, col)
    deg = dp[:NPAD, 0] + dp[NPAD:, 0] + 1.0
    deg = jnp.where(jnp.arange(NPAD) < N_NODES, deg, 1.0)
    dinve = lax.rsqrt(deg)[:, None]
    dsqrte = jnp.sqrt(deg)[:, None]
    dinv2e = (1.0 / deg)[:, None]

    h = _linear(x, lin1_w, lin1_b, relu=True)
    hs = jnp.pad(h, ((0, NPAD - N_NODES), (0, 0)))

    for coeffs in (coeffs0, coeffs1):
        z0 = dinve * hs
        acc = coeffs[0] * z0
        zc, zp = z0, z0
        for k in range(1, K_ORDER + 1):
            part = _prop(zc, row, col)
            if k == 1:
                a_n, c_n = 1.0, 0.0
            else:
                n = k - 1
                a_n = (2 * n + A_J + B_J + 1) * (2 * n + A_J + B_J + 2) / (
                    2 * (n + 1) * (n + A_J + B_J + 1))
                c_n = (n + A_J) * (n + B_J) * (2 * n + A_J + B_J + 2) / (
                    (n + 1) * (n + A_J + B_J + 1) * (2 * n + A_J + B_J))
            znew, acc = _step(part, zc, zp, acc, dinv2e,
                              coeffs[k].reshape(1, 1), a_n, c_n)
            zp, zc = zc, znew
        hs = jnp.maximum(dsqrte * acc, 0.0)

    out = _linear(hs[:N_NODES], lin2_w, lin2_b, relu=False)
    return out
